# baseline (speedup 1.0000x reference)
import jax
import jax.numpy as jnp
from jax.experimental import pallas as pl
from jax.experimental.pallas import tpu as pltpu

B, S, H, E = 4, 4096, 2048, 8
N = B * S
T = 256
NC = 16
KC = H // NC
TR = T * NC


def _router_block(xre_ref, ws_ref, mask_ref, logits_ref, idx_ref, w_ref):
    xre = xre_ref[...].astype(jnp.bfloat16)
    ws = ws_ref[...].astype(jnp.bfloat16)
    p = jax.lax.dot_general(ws, xre, (((1,), (1,)), ((), ())),
                            preferred_element_type=jnp.float32)
    p3 = p.reshape(128, T, NC)
    s2 = jnp.sum(p3 * mask_ref[...].reshape(128, 1, NC), axis=2)
    out8 = jnp.sum(s2.reshape(NC, E, T), axis=0)
    logits = out8.T
    logits_ref[...] = logits
    mx = jnp.max(logits, axis=1, keepdims=True)
    iota = jax.lax.broadcasted_iota(jnp.int32, logits.shape, 1)
    idx = jnp.min(jnp.where(logits == mx, iota, E), axis=1, keepdims=True)
    idx_ref[...] = idx
    w_ref[...] = jnp.ones_like(mx)


@jax.jit
def kernel(hidden_states, W):
    xre = hidden_states.reshape(N * NC, KC)
    ws = W.reshape(E, NC, KC).transpose(1, 0, 2).reshape(NC * E, KC)
    cidx = jax.lax.broadcasted_iota(jnp.int32, (NC * E, NC), 0) // E
    mask = (cidx == jax.lax.broadcasted_iota(jnp.int32, (NC * E, NC), 1))
    mask = mask.astype(jnp.float32)
    logits, idx, weights = pl.pallas_call(
        _router_block,
        grid=(N // T,),
        in_specs=[
            pl.BlockSpec((TR, KC), lambda i: (i, 0)),
            pl.BlockSpec((NC * E, KC), lambda i: (0, 0)),
            pl.BlockSpec((NC * E, NC), lambda i: (0, 0)),
        ],
        out_specs=[
            pl.BlockSpec((T, E), lambda i: (i, 0)),
            pl.BlockSpec((T, 1), lambda i: (i, 0)),
            pl.BlockSpec((T, 1), lambda i: (i, 0)),
        ],
        out_shape=[
            jax.ShapeDtypeStruct((N, E), jnp.float32),
            jax.ShapeDtypeStruct((N, 1), jnp.int32),
            jax.ShapeDtypeStruct((N, 1), jnp.float32),
        ],
        compiler_params=pltpu.CompilerParams(
            dimension_semantics=("parallel",),
            vmem_limit_bytes=110 * 1024 * 1024,
        ),
    )(xre, ws, mask)
    return (
        logits.reshape(B, S, E),
        idx.reshape(B, S),
        weights.reshape(B, S),
    )
